# baseline (speedup 1.0000x reference)
import jax
import jax.numpy as jnp
from jax import lax
from jax.experimental import pallas as pl
from jax.experimental.pallas import tpu as pltpu
from jax.experimental.pallas import tpu_sc as plsc

_NC = 2
_NS = 16
_NB = 2
_NO = 2
_C = 80
_K = 4
_D = 128
_L = 16
_NCHUNK = 1875
_QLO = _NCHUNK // (_NC * _NS)
_REM = _NCHUNK - _QLO * (_NC * _NS)
_QHI = _QLO + 1


def _body(x_hbm, c0_hbm, c1_hbm, c2_hbm, c3_hbm, out_hbm,
          v0, v1, v2, v3, r0, r1, o0, o1,
          g0, g1, s0, s1):
    cols_hbm = [c0_hbm, c1_hbm, c2_hbm, c3_hbm]
    colv = [v0, v1, v2, v3]
    rows = [r0, r1]
    outs = [o0, o1]
    gsem = [g0, g1]
    osem = [s0, s1]
    w = lax.axis_index("s") * _NC + lax.axis_index("c")
    base = w * _QLO + jnp.minimum(w, _REM)
    nw = _QLO + jnp.where(w < _REM, 1, 0)

    for k in range(_K):
        pltpu.sync_copy(cols_hbm[k].at[pl.ds(base * _C, _QLO * _C)],
                        colv[k].at[pl.ds(0, _QLO * _C)])

    @pl.when(w < _REM)
    def _():
        for k in range(_K):
            pltpu.sync_copy(
                cols_hbm[k].at[pl.ds((base + _QLO) * _C, _C)],
                colv[k].at[pl.ds(_QLO * _C, _C)])

    def gather_start(t, b):
        for k in range(_K):
            pltpu.async_copy(x_hbm.at[colv[k].at[pl.ds(t * _C, _C)]],
                             rows[b].at[pl.ds(k * _C, _C)], gsem[b])

    def gather_wait(b):
        pltpu.make_async_copy(x_hbm.at[pl.ds(0, _K * _C)], rows[b],
                              gsem[b]).wait()

    def out_wait(b):
        pltpu.make_async_copy(outs[b], out_hbm.at[pl.ds(0, _C)], osem[b]).wait()

    for b in range(_NB):
        gather_start(jnp.int32(b), b)

    @pl.loop(0, nw, step=_NB)
    def _group(g):
        for b in range(_NB):
            t = g + b

            ob = b % _NO

            @pl.when(t < nw)
            def _():
                gather_wait(b)

                @pl.when(t >= _NO)
                def _():
                    out_wait(ob)

                @pl.loop(0, _C)
                def _row(i):
                    for j in range(_D // _L):
                        s = pl.ds(j * _L, _L)
                        outs[ob][i, s] = ((rows[b][i, s]
                                          * rows[b][_C + i, s])
                                         * (rows[b][2 * _C + i, s]
                                            * rows[b][3 * _C + i, s]))

                @pl.when(t + _NB < nw)
                def _():
                    gather_start(t + _NB, b)

                pltpu.async_copy(outs[ob],
                                 out_hbm.at[pl.ds((base + t) * _C, _C)],
                                 osem[ob])

    for b in range(_NO):
        out_wait(b)


def kernel(x, indices):
    m, k = indices.shape
    d = x.shape[1]
    assert k == _K and d == _D and m == _NCHUNK * _C
    cols = [indices[:, kk] for kk in range(_K)]
    mesh = plsc.VectorSubcoreMesh(core_axis_name="c", subcore_axis_name="s")
    f = pl.kernel(
        _body,
        out_type=jax.ShapeDtypeStruct((m, d), jnp.float32),
        mesh=mesh,
        compiler_params=pltpu.CompilerParams(needs_layout_passes=False),
        scratch_types=(
            [pltpu.VMEM((_QHI * _C,), jnp.int32) for _ in range(_K)]
            + [pltpu.VMEM((_K * _C, _D), jnp.float32) for _ in range(_NB)]
            + [pltpu.VMEM((_C, _D), jnp.float32) for _ in range(_NO)]
            + [pltpu.SemaphoreType.DMA for _ in range(_NB + _NO)]
        ),
    )
    return f(x, *cols)


# speedup vs baseline: 1.0247x; 1.0247x over previous
import jax
import jax.numpy as jnp
from jax import lax
from jax.experimental import pallas as pl
from jax.experimental.pallas import tpu as pltpu
from jax.experimental.pallas import tpu_sc as plsc

_NC = 2
_NS = 16
_NB = 3
_NO = 3
_C = 48
_K = 4
_D = 128
_L = 16
_NCHUNK = 3125
_QLO = _NCHUNK // (_NC * _NS)
_REM = _NCHUNK - _QLO * (_NC * _NS)
_QHI = _QLO + 1


def _body(x_hbm, c0_hbm, c1_hbm, c2_hbm, c3_hbm, out_hbm,
          v0, v1, v2, v3, r0, r1, r2, o0, o1, o2,
          g0, g1, g2, s0, s1, s2):
    cols_hbm = [c0_hbm, c1_hbm, c2_hbm, c3_hbm]
    colv = [v0, v1, v2, v3]
    rows = [r0, r1, r2]
    outs = [o0, o1, o2]
    gsem = [g0, g1, g2]
    osem = [s0, s1, s2]
    w = lax.axis_index("s") * _NC + lax.axis_index("c")
    base = w * _QLO + jnp.minimum(w, _REM)
    nw = _QLO + jnp.where(w < _REM, 1, 0)

    for k in range(_K):
        pltpu.sync_copy(cols_hbm[k].at[pl.ds(base * _C, _QLO * _C)],
                        colv[k].at[pl.ds(0, _QLO * _C)])

    @pl.when(w < _REM)
    def _():
        for k in range(_K):
            pltpu.sync_copy(
                cols_hbm[k].at[pl.ds((base + _QLO) * _C, _C)],
                colv[k].at[pl.ds(_QLO * _C, _C)])

    def gather_start(t, b):
        for k in range(_K):
            pltpu.async_copy(x_hbm.at[colv[k].at[pl.ds(t * _C, _C)]],
                             rows[b].at[pl.ds(k * _C, _C)], gsem[b])

    def gather_wait(b):
        pltpu.make_async_copy(x_hbm.at[pl.ds(0, _K * _C)], rows[b],
                              gsem[b]).wait()

    def out_wait(b):
        pltpu.make_async_copy(outs[b], out_hbm.at[pl.ds(0, _C)], osem[b]).wait()

    for b in range(_NB):
        gather_start(jnp.int32(b), b)

    @pl.loop(0, nw, step=_NB)
    def _group(g):
        for b in range(_NB):
            t = g + b

            ob = b % _NO

            @pl.when(t < nw)
            def _():
                gather_wait(b)

                @pl.when(t >= _NO)
                def _():
                    out_wait(ob)

                @pl.loop(0, _C)
                def _row(i):
                    for j in range(_D // _L):
                        s = pl.ds(j * _L, _L)
                        outs[ob][i, s] = ((rows[b][i, s]
                                          * rows[b][_C + i, s])
                                         * (rows[b][2 * _C + i, s]
                                            * rows[b][3 * _C + i, s]))

                @pl.when(t + _NB < nw)
                def _():
                    gather_start(t + _NB, b)

                pltpu.async_copy(outs[ob],
                                 out_hbm.at[pl.ds((base + t) * _C, _C)],
                                 osem[ob])

    for b in range(_NO):
        out_wait(b)


def kernel(x, indices):
    m, k = indices.shape
    d = x.shape[1]
    assert k == _K and d == _D and m == _NCHUNK * _C
    cols = [indices[:, kk] for kk in range(_K)]
    mesh = plsc.VectorSubcoreMesh(core_axis_name="c", subcore_axis_name="s")
    f = pl.kernel(
        _body,
        out_type=jax.ShapeDtypeStruct((m, d), jnp.float32),
        mesh=mesh,
        compiler_params=pltpu.CompilerParams(needs_layout_passes=False),
        scratch_types=(
            [pltpu.VMEM((_QHI * _C,), jnp.int32) for _ in range(_K)]
            + [pltpu.VMEM((_K * _C, _D), jnp.float32) for _ in range(_NB)]
            + [pltpu.VMEM((_C, _D), jnp.float32) for _ in range(_NO)]
            + [pltpu.SemaphoreType.DMA for _ in range(_NB + _NO)]
        ),
    )
    return f(x, *cols)


# speedup vs baseline: 1.0405x; 1.0154x over previous
import jax
import jax.numpy as jnp
from jax import lax
from jax.experimental import pallas as pl
from jax.experimental.pallas import tpu as pltpu
from jax.experimental.pallas import tpu_sc as plsc

_NC = 2
_NS = 16
_NB = 3
_NO = 3
_C = 48
_K = 4
_D = 128
_L = 16
_NCHUNK = 3125
_QLO = _NCHUNK // (_NC * _NS)
_REM = _NCHUNK - _QLO * (_NC * _NS)
_QHI = _QLO + 1


def _body(x_hbm, c0_hbm, c1_hbm, c2_hbm, c3_hbm, out_hbm,
          v0, v1, v2, v3, r0, r1, r2, o0, o1, o2,
          g0, g1, g2, s0, s1, s2, ssem):
    cols_hbm = [c0_hbm, c1_hbm, c2_hbm, c3_hbm]
    colv = [v0, v1, v2, v3]
    rows = [r0, r1, r2]
    outs = [o0, o1, o2]
    gsem = [g0, g1, g2]
    osem = [s0, s1, s2]
    w = lax.axis_index("s") * _NC + lax.axis_index("c")
    base = w * _QLO + jnp.minimum(w, _REM)
    nw = _QLO + jnp.where(w < _REM, 1, 0)

    for k in range(_K):
        pltpu.async_copy(cols_hbm[k].at[pl.ds(base * _C, _QLO * _C)],
                         colv[k].at[pl.ds(0, _QLO * _C)], ssem)

    @pl.when(w < _REM)
    def _():
        for k in range(_K):
            pltpu.async_copy(
                cols_hbm[k].at[pl.ds((base + _QLO) * _C, _C)],
                colv[k].at[pl.ds(_QLO * _C, _C)], ssem)

    for k in range(_K):
        pltpu.make_async_copy(cols_hbm[k].at[pl.ds(0, _QLO * _C)],
                              colv[k].at[pl.ds(0, _QLO * _C)], ssem).wait()

    @pl.when(w < _REM)
    def _():
        for k in range(_K):
            pltpu.make_async_copy(
                cols_hbm[k].at[pl.ds(0, _C)],
                colv[k].at[pl.ds(_QLO * _C, _C)], ssem).wait()

    def gather_start(t, b):
        for k in range(_K):
            pltpu.async_copy(x_hbm.at[colv[k].at[pl.ds(t * _C, _C)]],
                             rows[b].at[pl.ds(k * _C, _C)], gsem[b])

    def gather_wait(b):
        pltpu.make_async_copy(x_hbm.at[pl.ds(0, _K * _C)], rows[b],
                              gsem[b]).wait()

    def out_wait(b):
        pltpu.make_async_copy(outs[b], out_hbm.at[pl.ds(0, _C)], osem[b]).wait()

    for b in range(_NB):
        gather_start(jnp.int32(b), b)

    @pl.loop(0, nw, step=_NB)
    def _group(g):
        for b in range(_NB):
            t = g + b

            ob = b % _NO

            @pl.when(t < nw)
            def _():
                gather_wait(b)

                @pl.when(t >= _NO)
                def _():
                    out_wait(ob)

                @pl.loop(0, _C)
                def _row(i):
                    for j in range(_D // _L):
                        s = pl.ds(j * _L, _L)
                        outs[ob][i, s] = ((rows[b][i, s]
                                          * rows[b][_C + i, s])
                                         * (rows[b][2 * _C + i, s]
                                            * rows[b][3 * _C + i, s]))

                @pl.when(t + _NB < nw)
                def _():
                    gather_start(t + _NB, b)

                pltpu.async_copy(outs[ob],
                                 out_hbm.at[pl.ds((base + t) * _C, _C)],
                                 osem[ob])

    for b in range(_NO):
        out_wait(b)


def kernel(x, indices):
    m, k = indices.shape
    d = x.shape[1]
    assert k == _K and d == _D and m == _NCHUNK * _C
    cols = [indices[:, kk] for kk in range(_K)]
    mesh = plsc.VectorSubcoreMesh(core_axis_name="c", subcore_axis_name="s")
    f = pl.kernel(
        _body,
        out_type=jax.ShapeDtypeStruct((m, d), jnp.float32),
        mesh=mesh,
        compiler_params=pltpu.CompilerParams(needs_layout_passes=False),
        scratch_types=(
            [pltpu.VMEM((_QHI * _C,), jnp.int32) for _ in range(_K)]
            + [pltpu.VMEM((_K * _C, _D), jnp.float32) for _ in range(_NB)]
            + [pltpu.VMEM((_C, _D), jnp.float32) for _ in range(_NO)]
            + [pltpu.SemaphoreType.DMA for _ in range(_NB + _NO + 1)]
        ),
    )
    return f(x, *cols)
